# Initial kernel scaffold; baseline (speedup 1.0000x reference)
import jax
import jax.numpy as jnp
from jax.experimental import pallas as pl


def kernel(words, tags, word_table, pos_table):
    raise NotImplementedError("write your pallas kernel here")


# speedup vs baseline: 1.9817x; 1.9817x over previous
import functools

import jax
import jax.numpy as jnp
from jax import lax
from jax.experimental import pallas as pl
from jax.experimental.pallas import tpu as pltpu
from jax.experimental.pallas import tpu_sc as plsc

NC, NS = 2, 16
NW = NC * NS
CHUNK = 128
LANES = 16


def kernel(words, tags, word_table, pos_table):
    B, L = words.shape
    D = word_table.shape[1]
    N = B * L
    n_per_w = N // NW
    n_chunks = n_per_w // CHUNK

    words_flat = words.reshape(N).astype(jnp.int32)
    tags_flat = tags.reshape(N).astype(jnp.int32)
    wtab128 = jnp.pad(word_table, ((0, 0), (0, D)))
    ptab128 = jnp.pad(pos_table, ((0, 0), (D, 0)))

    mesh = plsc.VectorSubcoreMesh(
        core_axis_name="c", subcore_axis_name="s",
        num_cores=NC, num_subcores=NS)

    @functools.partial(
        pl.kernel,
        out_type=jax.ShapeDtypeStruct((N, 2 * D), jnp.float32),
        mesh=mesh,
        scratch_types=[
            pltpu.VMEM((CHUNK,), jnp.int32),
            pltpu.VMEM((CHUNK,), jnp.int32),
            pltpu.VMEM((CHUNK, 2 * D), jnp.float32),
            pltpu.VMEM((CHUNK, 2 * D), jnp.float32),
            pltpu.SemaphoreType.DMA,
            pltpu.SemaphoreType.DMA,
        ],
    )
    def run(words_hbm, tags_hbm, wtab_hbm, ptab_hbm, out_hbm,
            widx, tidx, wrows, comb, sem_w, sem_p):
        wid = lax.axis_index("s") * NC + lax.axis_index("c")
        base0 = wid * n_per_w

        def body(i, carry):
            base = base0 + i * CHUNK
            pltpu.sync_copy(words_hbm.at[pl.ds(base, CHUNK)], widx)
            pltpu.sync_copy(tags_hbm.at[pl.ds(base, CHUNK)], tidx)
            cp = pltpu.async_copy(ptab_hbm.at[tidx], comb, sem_p)
            cw = pltpu.async_copy(wtab_hbm.at[widx], wrows, sem_w)
            cp.wait()
            cw.wait()

            def rep(r, c):
                for j in range(D // LANES):
                    comb[r, pl.ds(LANES * j, LANES)] = wrows[r, pl.ds(LANES * j, LANES)]
                return c

            lax.fori_loop(0, CHUNK, rep, 0)
            pltpu.sync_copy(comb, out_hbm.at[pl.ds(base, CHUNK), :])
            return carry

        lax.fori_loop(0, n_chunks, body, 0)

    out = run(words_flat, tags_flat, wtab128, ptab128)
    return out.reshape(B, L, 2 * D)
